# Initial kernel scaffold; baseline (speedup 1.0000x reference)
import jax
import jax.numpy as jnp
from jax.experimental import pallas as pl


def kernel(hidden_states, W_gate, W1, b1, W2, b2):
    raise NotImplementedError("write your pallas kernel here")


# speedup vs baseline: 5.4876x; 5.4876x over previous
import functools

import jax
import jax.numpy as jnp
from jax import lax
from jax.experimental import pallas as pl
from jax.experimental.pallas import tpu as pltpu
from jax.experimental.pallas import tpu_sc as plsc

NE = 8
TOPK = 2
H = 768
INTER = 3072
CAPF = 1.25

NW = 32
LANES = 16

BT = 256


def _gelu_exact(x):
    return 0.5 * x * (1.0 + lax.erf(x * (2.0 ** -0.5)))


def _gating_body(x_ref, wg_ref, e1_ref, e2_ref, g1_ref, g2_ref, aux_ref):
    x = x_ref[...]
    wg = wg_ref[...]
    logits = lax.dot_general(wg, x, (((1,), (1,)), ((), ())),
                             preferred_element_type=jnp.float32)
    t = logits.shape[1]
    m1 = logits[0]
    e1 = jnp.zeros((t,), jnp.int32)
    for e in range(1, NE):
        upd = logits[e] > m1
        m1 = jnp.where(upd, logits[e], m1)
        e1 = jnp.where(upd, e, e1)
    neg = jnp.float32(-jnp.inf)
    m2 = jnp.where(e1 == 0, neg, logits[0])
    e2 = jnp.zeros((t,), jnp.int32)
    for e in range(1, NE):
        le = jnp.where(e1 == e, neg, logits[e])
        upd = le > m2
        m2 = jnp.where(upd, le, m2)
        e2 = jnp.where(upd, e, e2)
    g1 = 1.0 / (1.0 + jnp.exp(m2 - m1))
    e1_ref[...] = e1
    e2_ref[...] = e2
    g1_ref[...] = g1
    g2_ref[...] = 1.0 - g1
    z = jnp.exp(logits - jnp.max(logits, axis=0, keepdims=True))
    probs = z / jnp.sum(z, axis=0, keepdims=True)
    mean_probs = jnp.mean(probs, axis=1)
    routing_probs = jnp.mean((probs > 0).astype(jnp.float32), axis=1)
    aux_ref[...] = (NE * jnp.sum(mean_probs * routing_probs)).reshape(1, 1)


def _gating(x, w_gate):
    t = x.shape[0]
    return pl.pallas_call(
        _gating_body,
        out_shape=[
            jax.ShapeDtypeStruct((t,), jnp.int32),
            jax.ShapeDtypeStruct((t,), jnp.int32),
            jax.ShapeDtypeStruct((t,), jnp.float32),
            jax.ShapeDtypeStruct((t,), jnp.float32),
            jax.ShapeDtypeStruct((1, 1), jnp.float32),
        ],
    )(x, w_gate)


def _routing_body(t, cap, seg, total, dummy,
                  e1_hbm, e2_hbm, g1_hbm, g2_hbm, x_hbm,
                  pos1_hbm, pos2_hbm, w1_hbm, w2_hbm, m_hbm, xp_hbm,
                  e1_v, e2_v, g1_v, g2_v, x_v,
                  r1_v, r2_v, p1g_v, p2g_v, p1s_v, p2s_v, w1_v, w2_v,
                  cnt_v, m0_v, m_v, sem):
    nc = 2
    wid = lax.axis_index("s") * nc + lax.axis_index("c")
    tpw = t // NW
    gpw = tpw // LANES
    ngrp = t // LANES
    base = wid * tpw
    mygrp = wid * gpw

    pltpu.sync_copy(e1_hbm, e1_v)
    pltpu.sync_copy(e2_hbm, e2_v)
    pltpu.sync_copy(g1_hbm.at[pl.ds(base, tpw)], g1_v)
    pltpu.sync_copy(g2_hbm.at[pl.ds(base, tpw)], g2_v)
    pltpu.sync_copy(x_hbm.at[pl.ds(base, tpw)], x_v)

    lane = lax.iota(jnp.int32, LANES)
    ones = jnp.ones((LANES,), jnp.int32)
    zeros = jnp.zeros((LANES,), jnp.int32)

    def rank_within(ids):
        rw = zeros
        for e in range(NE):
            m = ids == e
            cs = jnp.cumsum(m.astype(jnp.int32))
            rw = jnp.where(m, cs - 1, rw)
        return rw

    def hist(ids):
        h = zeros
        for e in range(NE):
            c = plsc.all_reduce_population_count(ids == e)
            h = h + jnp.where(lane == e, c, 0)
        return h

    def scan_body(g, carry):
        cnt0, cnt1 = carry
        ids1 = e1_v[pl.ds(g * LANES, LANES)]
        ids2 = e2_v[pl.ds(g * LANES, LANES)]
        in_mine = jnp.logical_and(g >= mygrp, g < mygrp + gpw)

        @pl.when(in_mine)
        def _():
            off = (g - mygrp) * LANES
            cnt_v[...] = cnt0
            b1 = plsc.load_gather(cnt_v, [ids1])
            r1_v[pl.ds(off, LANES)] = b1 + rank_within(ids1)
            cnt_v[...] = cnt1
            b2 = plsc.load_gather(cnt_v, [ids2])
            r2_v[pl.ds(off, LANES)] = b2 + rank_within(ids2)

        return cnt0 + hist(ids1), cnt1 + hist(ids2)

    tot0, tot1 = lax.fori_loop(0, ngrp, scan_body, (zeros, zeros))

    capv = jnp.full((LANES,), cap, jnp.int32)
    m0cap = jnp.minimum(tot0, capv)
    m0_v[...] = m0cap

    @pl.when(wid == 0)
    def _():
        m_v[...] = m0cap + jnp.minimum(tot1, capv)
        pltpu.sync_copy(m_v, m_hbm)

    def out_body(gg, _):
        off = gg * LANES
        ids1 = e1_v[pl.ds(base + off, LANES)]
        ids2 = e2_v[pl.ds(base + off, LANES)]
        r1 = r1_v[pl.ds(off, LANES)]
        r2 = r2_v[pl.ds(off, LANES)]
        k1 = r1 < cap
        k2 = r2 < cap
        p1 = ids1 * seg + r1
        p2 = ids2 * seg + plsc.load_gather(m0_v, [ids2]) + r2
        p1g_v[pl.ds(off, LANES)] = jnp.where(k1, p1, 0)
        p2g_v[pl.ds(off, LANES)] = jnp.where(k2, p2, 0)
        p1s_v[pl.ds(off, LANES)] = jnp.where(k1, p1, dummy)
        p2s_v[pl.ds(off, LANES)] = jnp.where(k2, p2, dummy)
        fz = jnp.zeros((LANES,), jnp.float32)
        w1_v[pl.ds(off, LANES)] = jnp.where(k1, g1_v[pl.ds(off, LANES)], fz)
        w2_v[pl.ds(off, LANES)] = jnp.where(k2, g2_v[pl.ds(off, LANES)], fz)
        return 0

    lax.fori_loop(0, gpw, out_body, 0)

    pltpu.sync_copy(p1g_v, pos1_hbm.at[pl.ds(base, tpw)])
    pltpu.sync_copy(p2g_v, pos2_hbm.at[pl.ds(base, tpw)])
    pltpu.sync_copy(w1_v, w1_hbm.at[pl.ds(base, tpw)])
    pltpu.sync_copy(w2_v, w2_hbm.at[pl.ds(base, tpw)])
    pltpu.async_copy(x_v, xp_hbm.at[p1s_v], sem).wait()
    pltpu.async_copy(x_v, xp_hbm.at[p2s_v], sem).wait()


def _routing(e1, e2, g1, g2, x, cap, seg, total, xp_rows):
    t = x.shape[0]
    tpw = t // NW
    mesh = plsc.VectorSubcoreMesh(core_axis_name="c", subcore_axis_name="s")
    body = functools.partial(_routing_body, t, cap, seg, total, total)
    kern = pl.kernel(
        body,
        out_type=[
            jax.ShapeDtypeStruct((t,), jnp.int32),
            jax.ShapeDtypeStruct((t,), jnp.int32),
            jax.ShapeDtypeStruct((t,), jnp.float32),
            jax.ShapeDtypeStruct((t,), jnp.float32),
            jax.ShapeDtypeStruct((LANES,), jnp.int32),
            jax.ShapeDtypeStruct((xp_rows, H), jnp.float32),
        ],
        mesh=mesh,
        scratch_types=[
            pltpu.VMEM((t,), jnp.int32),
            pltpu.VMEM((t,), jnp.int32),
            pltpu.VMEM((tpw,), jnp.float32),
            pltpu.VMEM((tpw,), jnp.float32),
            pltpu.VMEM((tpw, H), jnp.float32),
            pltpu.VMEM((tpw,), jnp.int32),
            pltpu.VMEM((tpw,), jnp.int32),
            pltpu.VMEM((tpw,), jnp.int32),
            pltpu.VMEM((tpw,), jnp.int32),
            pltpu.VMEM((tpw,), jnp.int32),
            pltpu.VMEM((tpw,), jnp.int32),
            pltpu.VMEM((tpw,), jnp.float32),
            pltpu.VMEM((tpw,), jnp.float32),
            pltpu.VMEM((LANES,), jnp.int32),
            pltpu.VMEM((LANES,), jnp.int32),
            pltpu.VMEM((LANES,), jnp.int32),
            pltpu.SemaphoreType.DMA,
        ],
        compiler_params=pltpu.CompilerParams(needs_layout_passes=False),
    )
    return kern(e1, e2, g1, g2, x)


def _ffn_body(m_ref, x_ref, w1_ref, b1_ref, w2_ref, b2_ref, y_ref):
    e = pl.program_id(0)
    j = pl.program_id(1)
    jl = (m_ref[e] + BT - 1) // BT

    @pl.when(j < jl)
    def _():
        xb = x_ref[...]
        h = lax.dot_general(xb, w1_ref[0], (((1,), (1,)), ((), ())),
                            preferred_element_type=jnp.float32)
        h = _gelu_exact(h + b1_ref[0])
        y = lax.dot_general(h, w2_ref[0], (((1,), (1,)), ((), ())),
                            preferred_element_type=jnp.float32)
        y_ref[...] = y + b2_ref[0]


def _ffn(xp, w1, b1, w2, b2, m, seg, total):
    nb = seg // BT

    def xmap(e, j, m_ref):
        jl = (m_ref[e] + BT - 1) // BT
        return (e * nb + jnp.minimum(j, jnp.maximum(jl - 1, 0)), 0)

    grid_spec = pltpu.PrefetchScalarGridSpec(
        num_scalar_prefetch=1,
        grid=(NE, nb),
        in_specs=[
            pl.BlockSpec((BT, H), xmap),
            pl.BlockSpec((1, INTER, H), lambda e, j, m_ref: (e, 0, 0)),
            pl.BlockSpec((1, 1, INTER), lambda e, j, m_ref: (e, 0, 0)),
            pl.BlockSpec((1, H, INTER), lambda e, j, m_ref: (e, 0, 0)),
            pl.BlockSpec((1, 1, H), lambda e, j, m_ref: (e, 0, 0)),
        ],
        out_specs=pl.BlockSpec((BT, H), xmap),
    )
    return pl.pallas_call(
        _ffn_body,
        grid_spec=grid_spec,
        out_shape=jax.ShapeDtypeStruct((total, H), jnp.float32),
    )(m, xp, w1, b1.reshape(NE, 1, INTER), w2, b2.reshape(NE, 1, H))


def _combine_body(t, y_hbm, p1_hbm, p2_hbm, w1_hbm, w2_hbm, out_hbm,
                  p1_v, p2_v, w1_v, w2_v, r1_v, r2_v, o_v, sem):
    nc = 2
    wid = lax.axis_index("s") * nc + lax.axis_index("c")
    tpw = t // NW
    base = wid * tpw
    rows = r1_v.shape[0]
    nch = tpw // rows

    pltpu.sync_copy(p1_hbm.at[pl.ds(base, tpw)], p1_v)
    pltpu.sync_copy(p2_hbm.at[pl.ds(base, tpw)], p2_v)
    pltpu.sync_copy(w1_hbm.at[pl.ds(base, tpw)], w1_v)
    pltpu.sync_copy(w2_hbm.at[pl.ds(base, tpw)], w2_v)

    fz = jnp.zeros((LANES,), jnp.float32)

    def chunk(c, _):
        pltpu.async_copy(y_hbm.at[p1_v.at[pl.ds(c * rows, rows)]], r1_v,
                         sem).wait()
        pltpu.async_copy(y_hbm.at[p2_v.at[pl.ds(c * rows, rows)]], r2_v,
                         sem).wait()

        def row(r, _):
            ridx = lax.broadcast(c * rows + r, (LANES,))
            w1b = plsc.load_gather(w1_v, [ridx])
            w2b = plsc.load_gather(w2_v, [ridx])
            z1 = w1b == 0.0
            z2 = w2b == 0.0
            for u in range(H // LANES):
                a = r1_v[r, pl.ds(u * LANES, LANES)]
                b = r2_v[r, pl.ds(u * LANES, LANES)]
                o = jnp.where(z1, fz, a * w1b) + jnp.where(z2, fz, b * w2b)
                o_v[r, pl.ds(u * LANES, LANES)] = o
            return 0

        lax.fori_loop(0, rows, row, 0)
        pltpu.sync_copy(o_v, out_hbm.at[pl.ds(base + c * rows, rows)])
        return 0

    lax.fori_loop(0, nch, chunk, 0)


def _combine(y, p1, p2, w1, w2, t):
    tpw = t // NW
    rows = 32
    mesh = plsc.VectorSubcoreMesh(core_axis_name="c", subcore_axis_name="s")
    kern = pl.kernel(
        functools.partial(_combine_body, t),
        out_type=jax.ShapeDtypeStruct((t, H), jnp.float32),
        mesh=mesh,
        scratch_types=[
            pltpu.VMEM((tpw,), jnp.int32),
            pltpu.VMEM((tpw,), jnp.int32),
            pltpu.VMEM((tpw,), jnp.float32),
            pltpu.VMEM((tpw,), jnp.float32),
            pltpu.VMEM((rows, H), jnp.float32),
            pltpu.VMEM((rows, H), jnp.float32),
            pltpu.VMEM((rows, H), jnp.float32),
            pltpu.SemaphoreType.DMA,
        ],
        compiler_params=pltpu.CompilerParams(needs_layout_passes=False),
    )
    return kern(y, p1, p2, w1, w2)


def kernel(hidden_states, W_gate, W1, b1, W2, b2):
    batch, seq, hidden = hidden_states.shape
    t = batch * seq
    cap = int(CAPF * seq * batch * TOPK / NE)
    seg = 2 * cap
    total = NE * seg
    xp_rows = total + BT

    x = hidden_states.reshape(t, hidden)
    e1, e2, g1, g2, aux = _gating(x, W_gate)
    p1, p2, w1z, w2z, m, xp = _routing(e1, e2, g1, g2, x, cap, seg, total,
                                       xp_rows)
    y = _ffn(xp, W1, b1, W2, b2, m[:NE], seg, total)
    out = _combine(y, p1, p2, w1z, w2z, t)
    return out.reshape(batch, seq, hidden), aux[0, 0]
